# Initial kernel scaffold; baseline (speedup 1.0000x reference)
import jax
import jax.numpy as jnp
from jax.experimental import pallas as pl


def kernel(x, edge_index, W_self, W_nbr, b):
    raise NotImplementedError("write your pallas kernel here")


# speedup vs baseline: 6.0083x; 6.0083x over previous
import functools

import jax
import jax.numpy as jnp
from jax import lax
from jax.experimental import pallas as pl
from jax.experimental.pallas import tpu as pltpu
from jax.experimental.pallas import tpu_sc as plsc

N_NODES = 10000
N_PAD = 10240
N_EDGES = 320000
D = 128

NC = 2
NS = 16
NW = NC * NS

CHUNK = 80
EDGES_PER_TILE = N_EDGES // NW
N_CHUNKS = EDGES_PER_TILE // CHUNK
COPY_ROWS = 624
DEG_COLS = N_PAD // NS
ZR = 16


def _sc_body(x_hbm, src_hbm, dst_hbm, agg_out, deg_out,
             agg_sh, stage_sh, src_idx, dst_idx, rows,
             hist_v, dega_v, degf_v, zeros_v, gsem):
    c = lax.axis_index("c")
    s = lax.axis_index("s")
    wid = c * NS + s
    base = wid * EDGES_PER_TILE

    def _fill_zeros(i, _):
        def _inner(j, _):
            zeros_v[i, pl.ds(j * 16, 16)] = jnp.zeros((16,), jnp.float32)
            return 0
        return lax.fori_loop(0, D // 16, _inner, 0)
    lax.fori_loop(0, ZR, _fill_zeros, 0)

    def _zero_hist(i, _):
        hist_v[pl.ds(i * 16, 16)] = jnp.zeros((16,), jnp.int32)
        return 0
    lax.fori_loop(0, N_PAD // 16, _zero_hist, 0)

    my_rows = jnp.where(s == NS - 1, N_NODES - (NS - 1) * COPY_ROWS, COPY_ROWS)
    row0 = s * COPY_ROWS

    def _zero(k, _):
        pltpu.sync_copy(zeros_v, agg_sh.at[pl.ds(row0 + k * ZR, ZR)])
        return 0
    lax.fori_loop(0, my_rows // ZR, _zero, 0)
    plsc.subcore_barrier()

    def _chunk(i, _):
        off = base + i * CHUNK
        pltpu.sync_copy(src_hbm.at[pl.ds(off, CHUNK)], src_idx.at[0])
        pltpu.sync_copy(dst_hbm.at[pl.ds(off, CHUNK)], dst_idx.at[0])
        pltpu.async_copy(x_hbm.at[src_idx.at[0]], rows.at[0], gsem).wait()
        pltpu.sync_copy(rows.at[0], agg_sh.at[dst_idx.at[0]], add=True)
        for j in range(CHUNK // 16):
            d = dst_idx[0, pl.ds(j * 16, 16)]
            cnt, last = plsc.scan_count(d)
            plsc.addupdate_scatter(hist_v, [d], cnt, mask=last)
        return 0
    lax.fori_loop(0, N_CHUNKS, _chunk, 0)

    pltpu.sync_copy(hist_v.at[pl.ds(0, N_PAD)], stage_sh.at[s])
    plsc.subcore_barrier()

    col0 = s * DEG_COLS
    pltpu.sync_copy(stage_sh.at[0, pl.ds(col0, DEG_COLS)],
                    hist_v.at[pl.ds(0, DEG_COLS)])

    def _combine(k, _):
        pltpu.sync_copy(stage_sh.at[k, pl.ds(col0, DEG_COLS)],
                        hist_v.at[pl.ds(DEG_COLS, DEG_COLS)])

        def _acc(t, _):
            a = hist_v[pl.ds(t * 16, 16)]
            b2 = hist_v[pl.ds(DEG_COLS + t * 16, 16)]
            hist_v[pl.ds(t * 16, 16)] = a + b2
            return 0
        return lax.fori_loop(0, DEG_COLS // 16, _acc, 0)
    lax.fori_loop(1, NS, _combine, 0)

    def _convert(t, _):
        degf_v[pl.ds(t * 16, 16)] = hist_v[pl.ds(t * 16, 16)].astype(jnp.float32)
        return 0
    lax.fori_loop(0, DEG_COLS // 16, _convert, 0)
    pltpu.sync_copy(degf_v, deg_out.at[pl.ds(c * N_PAD + col0, DEG_COLS)])

    pltpu.sync_copy(agg_sh.at[pl.ds(row0, my_rows)],
                    agg_out.at[c, pl.ds(row0, my_rows)])


@jax.jit
def _sc_aggregate(x, src, dst):
    mesh = plsc.VectorSubcoreMesh(core_axis_name="c", subcore_axis_name="s",
                                  num_cores=NC, num_subcores=NS)
    return pl.kernel(
        _sc_body,
        out_type=(
            jax.ShapeDtypeStruct((NC, N_NODES, D), jnp.float32),
            jax.ShapeDtypeStruct((NC * N_PAD,), jnp.float32),
        ),
        mesh=mesh,
        scratch_types=[
            pltpu.VMEM_SHARED((N_NODES, D), jnp.float32),
            pltpu.VMEM_SHARED((NS, N_PAD), jnp.int32),
            pltpu.VMEM((1, CHUNK), jnp.int32),
            pltpu.VMEM((1, CHUNK), jnp.int32),
            pltpu.VMEM((1, CHUNK, D), jnp.float32),
            pltpu.VMEM((N_PAD + DEG_COLS,), jnp.int32),
            pltpu.VMEM((DEG_COLS,), jnp.int32),
            pltpu.VMEM((DEG_COLS,), jnp.float32),
            pltpu.VMEM((ZR, D), jnp.float32),
            pltpu.SemaphoreType.DMA,
        ],
        compiler_params=pltpu.CompilerParams(needs_layout_passes=False),
    )(x, src, dst)


def _tc_body(x_ref, agg2_ref, deg2_ref, ws_ref, wn_ref, b_ref, out_ref):
    agg = agg2_ref[0] + agg2_ref[1]
    deg = deg2_ref[0] + deg2_ref[1]
    agg = agg / jnp.maximum(deg, 1.0)
    out = (jnp.dot(x_ref[...], ws_ref[...], preferred_element_type=jnp.float32)
           + jnp.dot(agg, wn_ref[...], preferred_element_type=jnp.float32)
           + b_ref[...][None, :])
    out_ref[...] = jnp.maximum(out, 0.0)


@jax.jit
def _tc_update(x, agg2, deg2, W_self, W_nbr, b):
    R = 1000
    grid = N_NODES // R
    return pl.pallas_call(
        _tc_body,
        grid=(grid,),
        in_specs=[
            pl.BlockSpec((R, D), lambda i: (i, 0)),
            pl.BlockSpec((NC, R, D), lambda i: (0, i, 0)),
            pl.BlockSpec((NC, R, 1), lambda i: (0, i, 0)),
            pl.BlockSpec((D, D), lambda i: (0, 0)),
            pl.BlockSpec((D, D), lambda i: (0, 0)),
            pl.BlockSpec((D,), lambda i: (0,)),
        ],
        out_specs=pl.BlockSpec((R, D), lambda i: (i, 0)),
        out_shape=jax.ShapeDtypeStruct((N_NODES, D), jnp.float32),
    )(x, agg2, deg2, W_self, W_nbr, b)


def kernel(x, edge_index, W_self, W_nbr, b):
    src = edge_index[0]
    dst = edge_index[1]
    agg2, deg_flat = _sc_aggregate(x, src, dst)
    deg2 = deg_flat.reshape(NC, N_PAD, 1)
    return _tc_update(x, agg2, deg2, W_self, W_nbr, b)


# speedup vs baseline: 12.1428x; 2.0210x over previous
import functools

import jax
import jax.numpy as jnp
from jax import lax
from jax.experimental import pallas as pl
from jax.experimental.pallas import tpu as pltpu
from jax.experimental.pallas import tpu_sc as plsc

N_NODES = 10000
N_PAD = 10240
N_EDGES = 320000
D = 128

NC = 2
NS = 16
NW = NC * NS

CHUNK = 80
EDGES_PER_TILE = N_EDGES // NW
N_CHUNKS = EDGES_PER_TILE // CHUNK
COPY_ROWS = 624
DEG_COLS = N_PAD // NS
ZR = 8


NB = 2
NIR = 3


def _sc_body(x_hbm, src_hbm, dst_hbm, agg_out, deg_out, hist_out,
             agg_sh, src_ring, dst_all, rows,
             hist_v, degf_v, zeros_v, gsems, ssems, isems, asem):
    c = lax.axis_index("c")
    s = lax.axis_index("s")
    wid = c * NS + s
    base = wid * EDGES_PER_TILE

    def _fill_zeros(i, _):
        def _inner(j, _):
            zeros_v[i, pl.ds(j * 16, 16)] = jnp.zeros((16,), jnp.float32)
            return 0
        return lax.fori_loop(0, D // 16, _inner, 0)
    lax.fori_loop(0, ZR, _fill_zeros, 0)

    def _zero_hist(i, _):
        hist_v[pl.ds(i * 16, 16)] = jnp.zeros((16,), jnp.int32)
        return 0
    lax.fori_loop(0, N_PAD // 16, _zero_hist, 0)

    my_rows = jnp.where(s == NS - 1, N_NODES - (NS - 1) * COPY_ROWS, COPY_ROWS)
    row0 = s * COPY_ROWS

    def _zero(k, _):
        pltpu.sync_copy(zeros_v, agg_sh.at[pl.ds(row0 + k * ZR, ZR)])
        return 0
    lax.fori_loop(0, my_rows // ZR, _zero, 0)

    pltpu.sync_copy(dst_hbm.at[wid], dst_all)
    plsc.subcore_barrier()

    gd = [None] * N_CHUNKS
    sd = [None] * N_CHUNKS
    idd = [None] * N_CHUNKS
    pltpu.sync_copy(src_hbm.at[pl.ds(base, CHUNK)], src_ring.at[0])
    if N_CHUNKS > 1:
        idd[1] = pltpu.async_copy(src_hbm.at[pl.ds(base + CHUNK, CHUNK)],
                                  src_ring.at[1], isems.at[1])
    gd[0] = pltpu.async_copy(x_hbm.at[src_ring.at[0]], rows.at[0], gsems.at[0])
    for i in range(N_CHUNKS):
        ip1, ip2 = i + 1, i + 2
        if ip1 < N_CHUNKS:
            idd[ip1].wait()
            if i >= 1:
                sd[i - 1].wait()
            gd[ip1] = pltpu.async_copy(x_hbm.at[src_ring.at[ip1 % NIR]],
                                       rows.at[ip1 % NB], gsems.at[ip1 % NB])
        if ip2 < N_CHUNKS:
            idd[ip2] = pltpu.async_copy(
                src_hbm.at[pl.ds(base + ip2 * CHUNK, CHUNK)],
                src_ring.at[ip2 % NIR], isems.at[ip2 % NIR])

        def _deg(jv, _, i=i):
            d = dst_all[i, pl.ds(jv * 16, 16)]
            cnt, last = plsc.scan_count(d)
            plsc.addupdate_scatter(hist_v, [d], cnt, mask=last)
            return 0
        lax.fori_loop(0, CHUNK // 16, _deg, 0)
        gd[i].wait()
        sd[i] = pltpu.async_copy(rows.at[i % NB], agg_sh.at[dst_all.at[i]],
                                 ssems.at[i % NB], add=True)
    sd[N_CHUNKS - 2].wait()
    sd[N_CHUNKS - 1].wait()

    pltpu.sync_copy(hist_v.at[pl.ds(0, N_PAD)],
                    hist_out.at[pl.ds(wid * N_PAD, N_PAD)])
    plsc.subcore_barrier()

    ad = pltpu.async_copy(agg_sh.at[pl.ds(row0, my_rows)],
                          agg_out.at[c, pl.ds(row0, my_rows)], asem)

    col0 = s * DEG_COLS
    core0 = c * NS * N_PAD
    tails = (N_PAD, N_PAD + DEG_COLS)
    pltpu.sync_copy(hist_out.at[pl.ds(core0 + col0, DEG_COLS)],
                    hist_v.at[pl.ds(0, DEG_COLS)])
    ld = [None] * NS
    ld[1] = pltpu.async_copy(
        hist_out.at[pl.ds(core0 + N_PAD + col0, DEG_COLS)],
        hist_v.at[pl.ds(tails[1 % 2], DEG_COLS)], isems.at[1 % 2])
    for k in range(1, NS):
        if k + 1 < NS:
            ld[k + 1] = pltpu.async_copy(
                hist_out.at[pl.ds(core0 + (k + 1) * N_PAD + col0, DEG_COLS)],
                hist_v.at[pl.ds(tails[(k + 1) % 2], DEG_COLS)],
                isems.at[(k + 1) % 2])
        ld[k].wait()

        def _acc(t, _, k=k):
            a = hist_v[pl.ds(t * 16, 16)]
            b2 = hist_v[pl.ds(tails[k % 2] + t * 16, 16)]
            hist_v[pl.ds(t * 16, 16)] = a + b2
            return 0
        lax.fori_loop(0, DEG_COLS // 16, _acc, 0)

    def _convert(t, _):
        degf_v[pl.ds(t * 16, 16)] = hist_v[pl.ds(t * 16, 16)].astype(jnp.float32)
        return 0
    lax.fori_loop(0, DEG_COLS // 16, _convert, 0)
    pltpu.sync_copy(degf_v, deg_out.at[pl.ds(c * N_PAD + col0, DEG_COLS)])
    ad.wait()


@jax.jit
def _sc_aggregate(x, src, dst):
    mesh = plsc.VectorSubcoreMesh(core_axis_name="c", subcore_axis_name="s",
                                  num_cores=NC, num_subcores=NS)
    return pl.kernel(
        _sc_body,
        out_type=(
            jax.ShapeDtypeStruct((NC, N_NODES, D), jnp.float32),
            jax.ShapeDtypeStruct((NC * N_PAD,), jnp.float32),
            jax.ShapeDtypeStruct((NW * N_PAD,), jnp.int32),
        ),
        mesh=mesh,
        scratch_types=[
            pltpu.VMEM_SHARED((N_NODES, D), jnp.float32),
            pltpu.VMEM((NIR, CHUNK), jnp.int32),
            pltpu.VMEM((N_CHUNKS, CHUNK), jnp.int32),
            pltpu.VMEM((NB, CHUNK, D), jnp.float32),
            pltpu.VMEM((N_PAD + 2 * DEG_COLS,), jnp.int32),
            pltpu.VMEM((DEG_COLS,), jnp.float32),
            pltpu.VMEM((ZR, D), jnp.float32),
            pltpu.SemaphoreType.DMA((NB,)),
            pltpu.SemaphoreType.DMA((NB,)),
            pltpu.SemaphoreType.DMA((NIR,)),
            pltpu.SemaphoreType.DMA,
        ],
        compiler_params=pltpu.CompilerParams(needs_layout_passes=False),
    )(x, src, dst)


def _tc_body(x_ref, agg2_ref, deg2_ref, ws_ref, wn_ref, b_ref, out_ref):
    agg = agg2_ref[0] + agg2_ref[1]
    deg = deg2_ref[0] + deg2_ref[1]
    agg = agg / jnp.maximum(deg, 1.0)
    out = (jnp.dot(x_ref[...], ws_ref[...], preferred_element_type=jnp.float32)
           + jnp.dot(agg, wn_ref[...], preferred_element_type=jnp.float32)
           + b_ref[...][None, :])
    out_ref[...] = jnp.maximum(out, 0.0)


@jax.jit
def _tc_update(x, agg2, deg2, W_self, W_nbr, b):
    R = 1000
    grid = N_NODES // R
    return pl.pallas_call(
        _tc_body,
        grid=(grid,),
        in_specs=[
            pl.BlockSpec((R, D), lambda i: (i, 0)),
            pl.BlockSpec((NC, R, D), lambda i: (0, i, 0)),
            pl.BlockSpec((NC, R, 1), lambda i: (0, i, 0)),
            pl.BlockSpec((D, D), lambda i: (0, 0)),
            pl.BlockSpec((D, D), lambda i: (0, 0)),
            pl.BlockSpec((D,), lambda i: (0,)),
        ],
        out_specs=pl.BlockSpec((R, D), lambda i: (i, 0)),
        out_shape=jax.ShapeDtypeStruct((N_NODES, D), jnp.float32),
    )(x, agg2, deg2, W_self, W_nbr, b)


def kernel(x, edge_index, W_self, W_nbr, b):
    src = edge_index[0]
    dst = edge_index[1].reshape(NW, N_CHUNKS, CHUNK)
    agg2, deg_flat, _ = _sc_aggregate(x, src, dst)
    deg2 = deg_flat.reshape(NC, N_PAD, 1)
    return _tc_update(x, agg2, deg2, W_self, W_nbr, b)


# speedup vs baseline: 12.1687x; 1.0021x over previous
import functools

import jax
import jax.numpy as jnp
from jax import lax
from jax.experimental import pallas as pl
from jax.experimental.pallas import tpu as pltpu
from jax.experimental.pallas import tpu_sc as plsc

N_NODES = 10000
N_PAD = 10240
N_EDGES = 320000
D = 128

NC = 2
NS = 16
NW = NC * NS

CHUNK = 80
EDGES_PER_TILE = N_EDGES // NW
N_CHUNKS = EDGES_PER_TILE // CHUNK
COPY_ROWS = 624
DEG_COLS = N_PAD // NS
ZR = 8


NB = 2
NIR = 3


def _sc_body(x_hbm, src_hbm, dst_hbm, agg_out, deg_out, hist_out,
             agg_sh, src_ring, dst_all, rows,
             hist_v, degf_v, zeros_v, gsems, ssems, isems, asem):
    c = lax.axis_index("c")
    s = lax.axis_index("s")
    wid = c * NS + s
    base = wid * EDGES_PER_TILE

    def _fill_zeros(i, _):
        def _inner(j, _):
            zeros_v[i, pl.ds(j * 16, 16)] = jnp.zeros((16,), jnp.float32)
            return 0
        return lax.fori_loop(0, D // 16, _inner, 0)
    lax.fori_loop(0, ZR, _fill_zeros, 0)

    def _zero_hist(i, _):
        hist_v[pl.ds(i * 16, 16)] = jnp.zeros((16,), jnp.int32)
        return 0
    lax.fori_loop(0, N_PAD // 16, _zero_hist, 0)

    my_rows = jnp.where(s == NS - 1, N_NODES - (NS - 1) * COPY_ROWS, COPY_ROWS)
    row0 = s * COPY_ROWS

    def _zero(k, _):
        pltpu.sync_copy(zeros_v, agg_sh.at[pl.ds(row0 + k * ZR, ZR)])
        return 0
    lax.fori_loop(0, my_rows // ZR, _zero, 0)

    pltpu.sync_copy(dst_hbm.at[wid], dst_all)
    plsc.subcore_barrier()

    gd = [None] * N_CHUNKS
    sd = [None] * N_CHUNKS
    idd = [None] * N_CHUNKS
    pltpu.sync_copy(src_hbm.at[pl.ds(base, CHUNK)], src_ring.at[0])
    if N_CHUNKS > 1:
        idd[1] = pltpu.async_copy(src_hbm.at[pl.ds(base + CHUNK, CHUNK)],
                                  src_ring.at[1], isems.at[1])
    gd[0] = pltpu.async_copy(x_hbm.at[src_ring.at[0]], rows.at[0], gsems.at[0])
    for i in range(N_CHUNKS):
        ip1, ip2 = i + 1, i + 2
        if ip1 < N_CHUNKS:
            idd[ip1].wait()
            if i >= 1:
                sd[i - 1].wait()
            gd[ip1] = pltpu.async_copy(x_hbm.at[src_ring.at[ip1 % NIR]],
                                       rows.at[ip1 % NB], gsems.at[ip1 % NB])
        if ip2 < N_CHUNKS:
            idd[ip2] = pltpu.async_copy(
                src_hbm.at[pl.ds(base + ip2 * CHUNK, CHUNK)],
                src_ring.at[ip2 % NIR], isems.at[ip2 % NIR])

        def _deg(jv, _, i=i):
            d = dst_all[i, pl.ds(jv * 16, 16)]
            cnt, last = plsc.scan_count(d)
            plsc.addupdate_scatter(hist_v, [d], cnt, mask=last)
            return 0
        lax.fori_loop(0, CHUNK // 16, _deg, 0)
        gd[i].wait()
        sd[i] = pltpu.async_copy(rows.at[i % NB], agg_sh.at[dst_all.at[i]],
                                 ssems.at[i % NB], add=True)
    sd[N_CHUNKS - 2].wait()
    sd[N_CHUNKS - 1].wait()

    pltpu.sync_copy(hist_v.at[pl.ds(0, N_PAD)],
                    hist_out.at[pl.ds(wid * N_PAD, N_PAD)])
    plsc.subcore_barrier()

    ad = pltpu.async_copy(agg_sh.at[pl.ds(row0, my_rows)],
                          agg_out.at[c, pl.ds(row0, my_rows)], asem)

    col0 = s * DEG_COLS
    core0 = c * NS * N_PAD
    tails = (N_PAD, N_PAD + DEG_COLS)
    pltpu.sync_copy(hist_out.at[pl.ds(core0 + col0, DEG_COLS)],
                    hist_v.at[pl.ds(0, DEG_COLS)])
    ld = [None] * NS
    ld[1] = pltpu.async_copy(
        hist_out.at[pl.ds(core0 + N_PAD + col0, DEG_COLS)],
        hist_v.at[pl.ds(tails[1 % 2], DEG_COLS)], isems.at[1 % 2])
    for k in range(1, NS):
        if k + 1 < NS:
            ld[k + 1] = pltpu.async_copy(
                hist_out.at[pl.ds(core0 + (k + 1) * N_PAD + col0, DEG_COLS)],
                hist_v.at[pl.ds(tails[(k + 1) % 2], DEG_COLS)],
                isems.at[(k + 1) % 2])
        ld[k].wait()

        def _acc(t, _, k=k):
            a = hist_v[pl.ds(t * 16, 16)]
            b2 = hist_v[pl.ds(tails[k % 2] + t * 16, 16)]
            hist_v[pl.ds(t * 16, 16)] = a + b2
            return 0
        lax.fori_loop(0, DEG_COLS // 16, _acc, 0)

    def _convert(t, _):
        degf_v[pl.ds(t * 16, 16)] = hist_v[pl.ds(t * 16, 16)].astype(jnp.float32)
        return 0
    lax.fori_loop(0, DEG_COLS // 16, _convert, 0)
    pltpu.sync_copy(degf_v, deg_out.at[pl.ds(c * N_PAD + col0, DEG_COLS)])
    ad.wait()


@jax.jit
def _sc_aggregate(x, src, dst):
    mesh = plsc.VectorSubcoreMesh(core_axis_name="c", subcore_axis_name="s",
                                  num_cores=NC, num_subcores=NS)
    return pl.kernel(
        _sc_body,
        out_type=(
            jax.ShapeDtypeStruct((NC, N_NODES, D), jnp.float32),
            jax.ShapeDtypeStruct((NC * N_PAD,), jnp.float32),
            jax.ShapeDtypeStruct((NW * N_PAD,), jnp.int32),
        ),
        mesh=mesh,
        scratch_types=[
            pltpu.VMEM_SHARED((N_NODES, D), jnp.float32),
            pltpu.VMEM((NIR, CHUNK), jnp.int32),
            pltpu.VMEM((N_CHUNKS, CHUNK), jnp.int32),
            pltpu.VMEM((NB, CHUNK, D), jnp.float32),
            pltpu.VMEM((N_PAD + 2 * DEG_COLS,), jnp.int32),
            pltpu.VMEM((DEG_COLS,), jnp.float32),
            pltpu.VMEM((ZR, D), jnp.float32),
            pltpu.SemaphoreType.DMA((NB,)),
            pltpu.SemaphoreType.DMA((NB,)),
            pltpu.SemaphoreType.DMA((NIR,)),
            pltpu.SemaphoreType.DMA,
        ],
        compiler_params=pltpu.CompilerParams(needs_layout_passes=False),
    )(x, src, dst)


def _tc_self_body(x_ref, ws_ref, b_ref, y_ref):
    y_ref[...] = (jnp.dot(x_ref[...], ws_ref[...],
                          preferred_element_type=jnp.float32)
                  + b_ref[...][None, :])


@jax.jit
def _tc_self(x, W_self, b):
    R = 1000
    return pl.pallas_call(
        _tc_self_body,
        grid=(N_NODES // R,),
        in_specs=[
            pl.BlockSpec((R, D), lambda i: (i, 0)),
            pl.BlockSpec((D, D), lambda i: (0, 0)),
            pl.BlockSpec((D,), lambda i: (0,)),
        ],
        out_specs=pl.BlockSpec((R, D), lambda i: (i, 0)),
        out_shape=jax.ShapeDtypeStruct((N_NODES, D), jnp.float32),
    )(x, W_self, b)


def _tc_body(y_ref, agg2_ref, deg2_ref, wn_ref, out_ref):
    agg = agg2_ref[0] + agg2_ref[1]
    deg = deg2_ref[0] + deg2_ref[1]
    agg = agg / jnp.maximum(deg, 1.0)
    out = y_ref[...] + jnp.dot(agg, wn_ref[...],
                               preferred_element_type=jnp.float32)
    out_ref[...] = jnp.maximum(out, 0.0)


@jax.jit
def _tc_update(y, agg2, deg2, W_nbr):
    R = 1000
    return pl.pallas_call(
        _tc_body,
        grid=(N_NODES // R,),
        in_specs=[
            pl.BlockSpec((R, D), lambda i: (i, 0)),
            pl.BlockSpec((NC, R, D), lambda i: (0, i, 0)),
            pl.BlockSpec((NC, R, 1), lambda i: (0, i, 0)),
            pl.BlockSpec((D, D), lambda i: (0, 0)),
        ],
        out_specs=pl.BlockSpec((R, D), lambda i: (i, 0)),
        out_shape=jax.ShapeDtypeStruct((N_NODES, D), jnp.float32),
    )(y, agg2, deg2, W_nbr)


def kernel(x, edge_index, W_self, W_nbr, b):
    src = edge_index[0]
    dst = edge_index[1].reshape(NW, N_CHUNKS, CHUNK)
    y = _tc_self(x, W_self, b)
    agg2, deg_flat, _ = _sc_aggregate(x, src, dst)
    deg2 = deg_flat.reshape(NC, N_PAD, 1)
    return _tc_update(y, agg2, deg2, W_nbr)


# speedup vs baseline: 12.2777x; 1.0090x over previous
import functools

import jax
import jax.numpy as jnp
from jax import lax
from jax.experimental import pallas as pl
from jax.experimental.pallas import tpu as pltpu
from jax.experimental.pallas import tpu_sc as plsc

N_NODES = 10000
N_PAD = 10240
N_EDGES = 320000
D = 128

NC = 2
NS = 16
NW = NC * NS

CHUNK = 80
EDGES_PER_TILE = N_EDGES // NW
N_CHUNKS = EDGES_PER_TILE // CHUNK
COPY_ROWS = 624
HIST_ROWS = 80
HIST_PAD = 128
ZR = 8


NB = 2
NIR = 3


def _sc_body(x_hbm, src_hbm, dst_hbm, agg_out, deg_out,
             agg_sh, deg_sh, src_ring, dst_all, rows,
             hist_v, conv_v, iota_v, zeros_v, gsems, ssems, isems, asem):
    c = lax.axis_index("c")
    s = lax.axis_index("s")
    wid = c * NS + s
    base = wid * EDGES_PER_TILE

    def _fill_zeros(i, _):
        def _inner(j, _):
            zeros_v[i, pl.ds(j * 16, 16)] = jnp.zeros((16,), jnp.float32)
            conv_v[i, pl.ds(j * 16, 16)] = jnp.zeros((16,), jnp.int32)
            return 0
        return lax.fori_loop(0, D // 16, _inner, 0)
    lax.fori_loop(0, ZR, _fill_zeros, 0)

    for j in range(HIST_ROWS // 16):
        iota_v[0, pl.ds(j * 16, 16)] = lax.iota(jnp.int32, 16) + (16 * j)

    def _zero_hist(i, _):
        def _inner(j, _):
            hist_v[i, pl.ds(j * 16, 16)] = jnp.zeros((16,), jnp.int32)
            return 0
        return lax.fori_loop(0, D // 16, _inner, 0)
    lax.fori_loop(0, HIST_ROWS, _zero_hist, 0)

    my_rows = jnp.where(s == NS - 1, N_NODES - (NS - 1) * COPY_ROWS, COPY_ROWS)
    row0 = s * COPY_ROWS
    allsems = ([gsems.at[k] for k in range(NB)]
               + [ssems.at[k] for k in range(NB)]
               + [isems.at[k] for k in range(NIR)] + [asem])
    zd = [pltpu.async_copy(zeros_v, agg_sh.at[pl.ds(row0 + k * ZR, ZR)],
                           allsems[k % len(allsems)])
          for k in range(COPY_ROWS // ZR)]
    pltpu.sync_copy(conv_v, deg_sh.at[pl.ds(s * ZR, ZR)])

    def _zero_rem(k, _):
        pltpu.sync_copy(zeros_v,
                        agg_sh.at[pl.ds(row0 + COPY_ROWS + k * ZR, ZR)])
        return 0
    lax.fori_loop(0, (my_rows - COPY_ROWS) // ZR, _zero_rem, 0)

    pltpu.sync_copy(dst_hbm.at[wid], dst_all)
    for d in zd:
        d.wait()
    plsc.subcore_barrier()

    gd = [None] * N_CHUNKS
    sd = [None] * N_CHUNKS
    idd = [None] * N_CHUNKS
    pltpu.sync_copy(src_hbm.at[pl.ds(base, CHUNK)], src_ring.at[0])
    if N_CHUNKS > 1:
        idd[1] = pltpu.async_copy(src_hbm.at[pl.ds(base + CHUNK, CHUNK)],
                                  src_ring.at[1], isems.at[1])
    gd[0] = pltpu.async_copy(x_hbm.at[src_ring.at[0]], rows.at[0], gsems.at[0])
    for i in range(N_CHUNKS):
        ip1, ip2 = i + 1, i + 2
        if ip1 < N_CHUNKS:
            idd[ip1].wait()
            if i >= 1:
                sd[i - 1].wait()
            gd[ip1] = pltpu.async_copy(x_hbm.at[src_ring.at[ip1 % NIR]],
                                       rows.at[ip1 % NB], gsems.at[ip1 % NB])
        if ip2 < N_CHUNKS:
            idd[ip2] = pltpu.async_copy(
                src_hbm.at[pl.ds(base + ip2 * CHUNK, CHUNK)],
                src_ring.at[ip2 % NIR], isems.at[ip2 % NIR])

        def _deg(jv, _, i=i):
            d = dst_all[i, pl.ds(jv * 16, 16)]
            cnt, last = plsc.scan_count(d)
            plsc.addupdate_scatter(hist_v, [d >> 7, d & 127], cnt, mask=last)
            return 0
        lax.fori_loop(0, CHUNK // 16, _deg, 0)
        gd[i].wait()
        sd[i] = pltpu.async_copy(rows.at[i % NB], agg_sh.at[dst_all.at[i]],
                                 ssems.at[i % NB], add=True)
    pltpu.sync_copy(hist_v, deg_sh.at[iota_v.at[0]], add=True)
    sd[N_CHUNKS - 2].wait()
    sd[N_CHUNKS - 1].wait()
    plsc.subcore_barrier()

    ad = pltpu.async_copy(agg_sh.at[pl.ds(row0, my_rows)],
                          agg_out.at[c, pl.ds(row0, my_rows)], asem)

    pltpu.sync_copy(deg_sh.at[pl.ds(s * ZR, ZR)], conv_v)

    def _convert(i, _):
        def _inner(j, _):
            zeros_v[i, pl.ds(j * 16, 16)] = (
                conv_v[i, pl.ds(j * 16, 16)].astype(jnp.float32))
            return 0
        return lax.fori_loop(0, D // 16, _inner, 0)
    lax.fori_loop(0, ZR, _convert, 0)
    pltpu.sync_copy(zeros_v, deg_out.at[pl.ds(c * HIST_PAD + s * ZR, ZR)])
    ad.wait()


@jax.jit
def _sc_aggregate(x, src, dst):
    mesh = plsc.VectorSubcoreMesh(core_axis_name="c", subcore_axis_name="s",
                                  num_cores=NC, num_subcores=NS)
    return pl.kernel(
        _sc_body,
        out_type=(
            jax.ShapeDtypeStruct((NC, N_NODES, D), jnp.float32),
            jax.ShapeDtypeStruct((NC * HIST_PAD, D), jnp.float32),
        ),
        mesh=mesh,
        scratch_types=[
            pltpu.VMEM_SHARED((N_NODES, D), jnp.float32),
            pltpu.VMEM_SHARED((HIST_PAD, D), jnp.int32),
            pltpu.VMEM((NIR, CHUNK), jnp.int32),
            pltpu.VMEM((N_CHUNKS, CHUNK), jnp.int32),
            pltpu.VMEM((NB, CHUNK, D), jnp.float32),
            pltpu.VMEM((HIST_ROWS, D), jnp.int32),
            pltpu.VMEM((ZR, D), jnp.int32),
            pltpu.VMEM((1, HIST_ROWS), jnp.int32),
            pltpu.VMEM((ZR, D), jnp.float32),
            pltpu.SemaphoreType.DMA((NB,)),
            pltpu.SemaphoreType.DMA((NB,)),
            pltpu.SemaphoreType.DMA((NIR,)),
            pltpu.SemaphoreType.DMA,
        ],
        compiler_params=pltpu.CompilerParams(needs_layout_passes=False),
    )(x, src, dst)


def _tc_self_body(x_ref, ws_ref, b_ref, y_ref):
    y_ref[...] = (jnp.dot(x_ref[...], ws_ref[...],
                          preferred_element_type=jnp.float32)
                  + b_ref[...][None, :])


@jax.jit
def _tc_self(x, W_self, b):
    R = 1000
    return pl.pallas_call(
        _tc_self_body,
        grid=(N_NODES // R,),
        in_specs=[
            pl.BlockSpec((R, D), lambda i: (i, 0)),
            pl.BlockSpec((D, D), lambda i: (0, 0)),
            pl.BlockSpec((D,), lambda i: (0,)),
        ],
        out_specs=pl.BlockSpec((R, D), lambda i: (i, 0)),
        out_shape=jax.ShapeDtypeStruct((N_NODES, D), jnp.float32),
    )(x, W_self, b)


def _tc_body(y_ref, agg2_ref, deg2_ref, wn_ref, out_ref):
    agg = agg2_ref[0] + agg2_ref[1]
    deg = deg2_ref[0] + deg2_ref[1]
    agg = agg / jnp.maximum(deg, 1.0)
    out = y_ref[...] + jnp.dot(agg, wn_ref[...],
                               preferred_element_type=jnp.float32)
    out_ref[...] = jnp.maximum(out, 0.0)


@jax.jit
def _tc_update(y, agg2, deg2, W_nbr):
    R = 1000
    return pl.pallas_call(
        _tc_body,
        grid=(N_NODES // R,),
        in_specs=[
            pl.BlockSpec((R, D), lambda i: (i, 0)),
            pl.BlockSpec((NC, R, D), lambda i: (0, i, 0)),
            pl.BlockSpec((NC, R, 1), lambda i: (0, i, 0)),
            pl.BlockSpec((D, D), lambda i: (0, 0)),
        ],
        out_specs=pl.BlockSpec((R, D), lambda i: (i, 0)),
        out_shape=jax.ShapeDtypeStruct((N_NODES, D), jnp.float32),
    )(y, agg2, deg2, W_nbr)


def kernel(x, edge_index, W_self, W_nbr, b):
    src = edge_index[0]
    dst = edge_index[1].reshape(NW, N_CHUNKS, CHUNK)
    y = _tc_self(x, W_self, b)
    agg2, deg_flat = _sc_aggregate(x, src, dst)
    deg2 = deg_flat.reshape(NC, HIST_PAD * D, 1)
    return _tc_update(y, agg2, deg2, W_nbr)
